# Initial kernel scaffold; baseline (speedup 1.0000x reference)
import jax
import jax.numpy as jnp
from jax.experimental import pallas as pl


def kernel(node_indices_batch, edge_index_batch, edge_weights_batch, node_graph_ids_batch, emb_table, edge_params, proj_W, proj_b, clf_W, clf_b, fusion_W, fusion_b):
    raise NotImplementedError("write your pallas kernel here")


# speedup vs baseline: 3.6779x; 3.6779x over previous
import functools

import jax
import jax.numpy as jnp
from jax import lax
from jax.experimental import pallas as pl
from jax.experimental.pallas import tpu as pltpu
from jax.experimental.pallas import tpu_sc as plsc

NG = 4
N = 10000
NP = 10240
E = 320000
B = 16
VOCAB = 100000
D = 128
C = 50
HEADS = 3

NC = 2
NS = 16
NW = NC * NS

GCH = 640
EPT = E // NS
K = 400
NCH = EPT // K
NPT = NP // NS

_mesh = plsc.VectorSubcoreMesh(core_axis_name="c", subcore_axis_name="s")


@functools.partial(
    pl.kernel,
    out_type=(
        jax.ShapeDtypeStruct((NG * NP, D), jnp.float32),
        jax.ShapeDtypeStruct((NG * E,), jnp.float32),
        jax.ShapeDtypeStruct((NG * NP,), jnp.float32),
    ),
    mesh=_mesh,
    scratch_types=[
        pltpu.VMEM((GCH,), jnp.int32),
        pltpu.VMEM((GCH, D), jnp.float32),
        pltpu.VMEM((1, K), jnp.int32),
        pltpu.VMEM((K,), jnp.float32),
        pltpu.VMEM((K,), jnp.float32),
        pltpu.VMEM((16,), jnp.float32),
        pltpu.VMEM((K,), jnp.float32),
        pltpu.SemaphoreType.DMA,
        pltpu.VMEM_SHARED((NP,), jnp.float32),
    ],
)
def _sc_prep(ni_hbm, emb_hbm, dst_hbm, ew_hbm, params_hbm, zn_hbm,
             x_out, ewp_out, deg_out,
             idxv, xrows, dstv, ewv, ewpv, pv, onesv, sem, deg_sh):
    c = lax.axis_index("c")
    s = lax.axis_index("s")
    w = s * NC + c

    base = w * (2 * GCH)
    for j in range(2):
        pltpu.sync_copy(ni_hbm.at[pl.ds(base + j * GCH, GCH)], idxv)
        pltpu.async_copy(emb_hbm.at[idxv], xrows, sem).wait()
        pltpu.sync_copy(xrows, x_out.at[pl.ds(base + j * GCH, GCH)])

    for i in range(K // 16):
        onesv[pl.ds(i * 16, 16)] = jnp.full((16,), 1.0, jnp.float32)

    for gi in range(2):
        g = 2 * c + gi
        pltpu.sync_copy(zn_hbm.at[pl.ds(s * NPT, NPT)],
                        deg_sh.at[pl.ds(s * NPT, NPT)])
        plsc.subcore_barrier()

        ebase = g * E + s * EPT

        def deg_body(i, _):
            pltpu.sync_copy(dst_hbm.at[pl.ds(ebase + i * K, K)], dstv.at[0])
            pltpu.sync_copy(onesv, deg_sh.at[dstv.at[0]], add=True)
            return 0
        lax.fori_loop(0, NCH, deg_body, 0)
        plsc.subcore_barrier()

        pltpu.sync_copy(deg_sh.at[pl.ds(s * NPT, NPT)],
                        deg_out.at[pl.ds(g * NP + s * NPT, NPT)])

        pltpu.sync_copy(params_hbm.at[pl.ds(g * 16, 16)], pv)

        def ew_body(i, _):
            pltpu.sync_copy(ew_hbm.at[pl.ds(ebase + i * K, K)], ewv)
            for k in range(K // 16):
                w16 = ewv[pl.ds(k * 16, 16)]
                ewpv[pl.ds(k * 16, 16)] = jnp.maximum(w16 * pv[...], 0.0)
            pltpu.sync_copy(ewpv, ewp_out.at[pl.ds(ebase + i * K, K)])
            return 0
        lax.fori_loop(0, NCH, ew_body, 0)
        plsc.subcore_barrier()


DH = D // 2
HCH = NPT // 2


@functools.partial(
    pl.kernel,
    out_type=(
        jax.ShapeDtypeStruct((NG * NP, DH), jnp.float32),
        jax.ShapeDtypeStruct((NG * NP, DH), jnp.float32),
        jax.ShapeDtypeStruct((NG * NP, DH), jnp.float32),
        jax.ShapeDtypeStruct((NG * NP, DH), jnp.float32),
    ),
    mesh=_mesh,
    scratch_types=[
        pltpu.VMEM((1, K), jnp.int32),
        pltpu.VMEM((1, K), jnp.int32),
        pltpu.VMEM((K,), jnp.float32),
        pltpu.VMEM((K, DH), jnp.float32),
        pltpu.VMEM((HCH, DH), jnp.float32),
        pltpu.VMEM((NPT,), jnp.float32),
        pltpu.SemaphoreType.DMA,
        pltpu.VMEM_SHARED((NP, DH), jnp.float32),
    ],
    compiler_params=pltpu.CompilerParams(use_tc_tiling_on_sc=False),
)
def _sc_msgpass(h0a_hbm, h0b_hbm, src_hbm, dst_hbm, ewp_hbm, deg_hbm, znd_hbm,
                h1a_out, h1b_out, h2a_out, h2b_out,
                srcv, dstv, ewv, rows, hbuf, invv, sem, agg_sh):
    c = lax.axis_index("c")
    s = lax.axis_index("s")

    for gi in range(2):
        g = 2 * c + gi
        nbase = g * NP + s * NPT
        ebase = g * E + s * EPT

        pltpu.sync_copy(deg_hbm.at[pl.ds(nbase, NPT)], invv)

        def inv_body(i, _):
            d16 = invv[pl.ds(i * 16, 16)]
            invv[pl.ds(i * 16, 16)] = 1.0 / jnp.maximum(d16, 1.0)
            return 0
        lax.fori_loop(0, NPT // 16, inv_body, 0)

        for step in range(2):
            halves_src = (h0a_hbm, h0b_hbm) if step == 0 else (h1a_out, h1b_out)
            halves_dst = (h1a_out, h1b_out) if step == 0 else (h2a_out, h2b_out)
            for fi in range(2):
                h_src = halves_src[fi]
                h_dst = halves_dst[fi]

                pltpu.sync_copy(znd_hbm.at[pl.ds(s * NPT, NPT)],
                                agg_sh.at[pl.ds(s * NPT, NPT)])
                plsc.subcore_barrier()

                def body(i, _):
                    pltpu.sync_copy(src_hbm.at[pl.ds(ebase + i * K, K)],
                                    srcv.at[0])
                    pltpu.sync_copy(dst_hbm.at[pl.ds(ebase + i * K, K)],
                                    dstv.at[0])
                    pltpu.sync_copy(ewp_hbm.at[pl.ds(ebase + i * K, K)], ewv)
                    pltpu.async_copy(h_src.at[srcv.at[0]], rows, sem).wait()

                    def mul_body(k16, _):
                        kb = k16 * 16
                        w16 = ewv[pl.ds(kb, 16)]
                        for j in range(16):
                            wk = w16[j]
                            for d in range(DH // 16):
                                rows[kb + j, pl.ds(d * 16, 16)] = (
                                    rows[kb + j, pl.ds(d * 16, 16)] * wk)
                        return 0
                    lax.fori_loop(0, K // 16, mul_body, 0)

                    pltpu.sync_copy(rows, agg_sh.at[dstv.at[0]], add=True)
                    return 0
                lax.fori_loop(0, NCH, body, 0)
                plsc.subcore_barrier()

                for r in range(2):
                    rb = r * HCH
                    pltpu.sync_copy(agg_sh.at[pl.ds(s * NPT + rb, HCH)],
                                    rows.at[pl.ds(0, HCH)])
                    pltpu.sync_copy(h_src.at[pl.ds(nbase + rb, HCH)], hbuf)

                    def upd_body(i2, _):
                        ib = i2 * 16
                        iv16 = invv[pl.ds(rb + ib, 16)]
                        for j in range(16):
                            sj = iv16[j]
                            for d in range(DH // 16):
                                hbuf[ib + j, pl.ds(d * 16, 16)] = (
                                    hbuf[ib + j, pl.ds(d * 16, 16)]
                                    + rows[ib + j, pl.ds(d * 16, 16)] * sj)
                        return 0
                    lax.fori_loop(0, HCH // 16, upd_body, 0)
                    pltpu.sync_copy(hbuf, h_dst.at[pl.ds(nbase + rb, HCH)])
                plsc.subcore_barrier()


_PBLK = 1024


def _tc_proj(x, w, b):
    def body(x_ref, w_ref, b_ref, oa_ref, ob_ref):
        h = (jnp.dot(x_ref[0], w_ref[0], preferred_element_type=jnp.float32)
             + b_ref[0])
        oa_ref[0] = h[:, :DH]
        ob_ref[0] = h[:, DH:]
    return pl.pallas_call(
        body,
        grid=(NG, NP // _PBLK),
        in_specs=[
            pl.BlockSpec((1, _PBLK, D), lambda g, nb: (g, nb, 0)),
            pl.BlockSpec((1, D, D), lambda g, nb: (g, 0, 0)),
            pl.BlockSpec((1, 1, D), lambda g, nb: (g, 0, 0)),
        ],
        out_specs=[
            pl.BlockSpec((1, _PBLK, DH), lambda g, nb: (g, nb, 0)),
            pl.BlockSpec((1, _PBLK, DH), lambda g, nb: (g, nb, 0)),
        ],
        out_shape=[
            jax.ShapeDtypeStruct((NG, NP, DH), jnp.float32),
            jax.ShapeDtypeStruct((NG, NP, DH), jnp.float32),
        ],
    )(x, w, b)


def _tc_readout(h2in, seg3, cw, cb):
    def body(h_ref, s_ref, w_ref, b_ref, o_ref):
        h2 = h_ref[0]
        seg = s_ref[0]
        io = lax.broadcasted_iota(jnp.int32, (NP, B), 1)
        onehot = (seg == io).astype(jnp.float32)
        counts = jnp.sum(onehot, axis=0)
        doc = lax.dot_general(onehot, h2, (((0,), (0,)), ((), ())))
        doc = doc / jnp.maximum(counts, 1.0)[:, None]
        a = jnp.maximum(doc, 0.0)
        o_ref[0] = (
            jnp.dot(a, w_ref[0], preferred_element_type=jnp.float32)
            + b_ref[0]
        )
    return pl.pallas_call(
        body,
        grid=(NG,),
        in_specs=[
            pl.BlockSpec((1, NP, D), lambda g: (g, 0, 0)),
            pl.BlockSpec((1, NP, 1), lambda g: (g, 0, 0)),
            pl.BlockSpec((1, D, C), lambda g: (g, 0, 0)),
            pl.BlockSpec((1, 1, C), lambda g: (g, 0, 0)),
        ],
        out_specs=pl.BlockSpec((1, B, C), lambda g: (g, 0, 0)),
        out_shape=jax.ShapeDtypeStruct((NG, B, C), jnp.float32),
    )(h2in, seg3, cw, cb)


def _tc_head(logits, fw, fb):
    def body(l_ref, w_ref, b_ref, p_ref, f_ref):
        wm = jnp.mean(w_ref[...], axis=0)
        bm = jnp.mean(b_ref[...], axis=0)
        acc = jnp.zeros((B, C), jnp.float32)
        for g in range(NG):
            acc = acc + lax.dot_general(
                l_ref[g], wm[:, :, g], (((1,), (1,)), ((), ())),
                preferred_element_type=jnp.float32)
        fused = acc + bm[None, :]
        m = jnp.max(fused, axis=-1, keepdims=True)
        e = jnp.exp(fused - m)
        p = e / jnp.sum(e, axis=-1, keepdims=True)
        p_ref[...] = p
        f_ref[...] = fused
    return pl.pallas_call(
        body,
        in_specs=[
            pl.BlockSpec(logits.shape, lambda: (0, 0, 0)),
            pl.BlockSpec(fw.shape, lambda: (0, 0, 0, 0)),
            pl.BlockSpec(fb.shape, lambda: (0, 0)),
        ],
        out_specs=[
            pl.BlockSpec((B, C), lambda: (0, 0)),
            pl.BlockSpec((B, C), lambda: (0, 0)),
        ],
        out_shape=[
            jax.ShapeDtypeStruct((B, C), jnp.float32),
            jax.ShapeDtypeStruct((B, C), jnp.float32),
        ],
    )(logits, fw, fb)


def kernel(node_indices_batch, edge_index_batch, edge_weights_batch,
           node_graph_ids_batch, emb_table, edge_params, proj_W, proj_b,
           clf_W, clf_b, fusion_W, fusion_b):
    ni_pad = jnp.concatenate(
        [node_indices_batch,
         jnp.zeros((NG, NP - N), jnp.int32)], axis=1).reshape(-1)
    src = edge_index_batch[:, 0, :]
    dst = edge_index_batch[:, 1, :]
    src_glob = (src + (jnp.arange(NG, dtype=jnp.int32) * NP)[:, None]).reshape(-1)
    dst_flat = dst.reshape(-1)
    ew_flat = edge_weights_batch.reshape(-1)
    params_rep = jnp.broadcast_to(edge_params[:, None], (NG, 16)).reshape(-1)
    zeros_n = jnp.zeros((NP,), jnp.float32)
    zeros_nd = jnp.zeros((NP, DH), jnp.float32)
    seg3 = jnp.concatenate(
        [node_graph_ids_batch,
         jnp.full((NG, NP - N), B, jnp.int32)],
        axis=1)[:, :, None]

    x_flat, ewp_flat, deg_flat = _sc_prep(ni_pad, emb_table, dst_flat, ew_flat,
                                          params_rep, zeros_n)

    h0a, h0b = _tc_proj(x_flat.reshape(NG, NP, D), proj_W,
                        proj_b[:, None, :])

    _, _, h2a, h2b = _sc_msgpass(h0a.reshape(NG * NP, DH),
                                 h0b.reshape(NG * NP, DH),
                                 src_glob, dst_flat, ewp_flat, deg_flat,
                                 zeros_nd)
    h2 = jnp.concatenate([h2a, h2b], axis=1)

    logits = _tc_readout(h2.reshape(NG, NP, D), seg3, clf_W,
                         clf_b[:, None, :])

    predictions, fused_logits = _tc_head(logits, fusion_W, fusion_b)
    return (predictions, fused_logits)


# speedup vs baseline: 4.5933x; 1.2489x over previous
import functools

import jax
import jax.numpy as jnp
from jax import lax
from jax.experimental import pallas as pl
from jax.experimental.pallas import tpu as pltpu
from jax.experimental.pallas import tpu_sc as plsc

NG = 4
N = 10000
NP = 10240
E = 320000
B = 16
VOCAB = 100000
D = 128
C = 50
HEADS = 3

NC = 2
NS = 16
NW = NC * NS

GCH = 640
EPT = E // NS
K = 400
NCH = EPT // K
NPT = NP // NS

_mesh = plsc.VectorSubcoreMesh(core_axis_name="c", subcore_axis_name="s")


@functools.partial(
    pl.kernel,
    out_type=(
        jax.ShapeDtypeStruct((NG * NP, D), jnp.float32),
        jax.ShapeDtypeStruct((NG * E,), jnp.float32),
        jax.ShapeDtypeStruct((NG * NP,), jnp.float32),
    ),
    mesh=_mesh,
    scratch_types=[
        pltpu.VMEM((GCH,), jnp.int32),
        pltpu.VMEM((GCH, D), jnp.float32),
        pltpu.VMEM((1, K), jnp.int32),
        pltpu.VMEM((K,), jnp.float32),
        pltpu.VMEM((K,), jnp.float32),
        pltpu.VMEM((16,), jnp.float32),
        pltpu.VMEM((K,), jnp.float32),
        pltpu.SemaphoreType.DMA,
        pltpu.VMEM_SHARED((NP,), jnp.float32),
    ],
)
def _sc_prep(ni_hbm, emb_hbm, dst_hbm, ew_hbm, params_hbm, zn_hbm,
             x_out, ewp_out, deg_out,
             idxv, xrows, dstv, ewv, ewpv, pv, onesv, sem, deg_sh):
    c = lax.axis_index("c")
    s = lax.axis_index("s")
    w = s * NC + c

    base = w * (2 * GCH)
    for j in range(2):
        pltpu.sync_copy(ni_hbm.at[pl.ds(base + j * GCH, GCH)], idxv)
        pltpu.async_copy(emb_hbm.at[idxv], xrows, sem).wait()
        pltpu.sync_copy(xrows, x_out.at[pl.ds(base + j * GCH, GCH)])

    for i in range(K // 16):
        onesv[pl.ds(i * 16, 16)] = jnp.full((16,), 1.0, jnp.float32)

    for gi in range(2):
        g = 2 * c + gi
        pltpu.sync_copy(zn_hbm.at[pl.ds(s * NPT, NPT)],
                        deg_sh.at[pl.ds(s * NPT, NPT)])
        plsc.subcore_barrier()

        ebase = g * E + s * EPT

        def deg_body(i, _):
            pltpu.sync_copy(dst_hbm.at[pl.ds(ebase + i * K, K)], dstv.at[0])
            pltpu.sync_copy(onesv, deg_sh.at[dstv.at[0]], add=True)
            return 0
        lax.fori_loop(0, NCH, deg_body, 0)
        plsc.subcore_barrier()

        pltpu.sync_copy(deg_sh.at[pl.ds(s * NPT, NPT)],
                        deg_out.at[pl.ds(g * NP + s * NPT, NPT)])

        pltpu.sync_copy(params_hbm.at[pl.ds(g * 16, 16)], pv)

        def ew_body(i, _):
            pltpu.sync_copy(ew_hbm.at[pl.ds(ebase + i * K, K)], ewv)
            for k in range(K // 16):
                w16 = ewv[pl.ds(k * 16, 16)]
                ewpv[pl.ds(k * 16, 16)] = jnp.maximum(w16 * pv[...], 0.0)
            pltpu.sync_copy(ewpv, ewp_out.at[pl.ds(ebase + i * K, K)])
            return 0
        lax.fori_loop(0, NCH, ew_body, 0)
        plsc.subcore_barrier()


DH = D // 2
HCH = NPT // 2


@functools.partial(
    pl.kernel,
    out_type=(
        jax.ShapeDtypeStruct((2 * NG * NP, DH), jnp.float32),
        jax.ShapeDtypeStruct((2 * NG * NP, DH), jnp.float32),
    ),
    mesh=_mesh,
    scratch_types=[
        pltpu.VMEM((1, K), jnp.int32),
        pltpu.VMEM((1, K), jnp.int32),
        pltpu.VMEM((K,), jnp.float32),
        pltpu.VMEM((1, K), jnp.int32),
        pltpu.VMEM((1, K), jnp.int32),
        pltpu.VMEM((K,), jnp.float32),
        pltpu.VMEM((K, DH), jnp.float32),
        pltpu.VMEM((K, DH), jnp.float32),
        pltpu.VMEM((NPT,), jnp.float32),
        pltpu.SemaphoreType.DMA,
        pltpu.SemaphoreType.DMA,
        pltpu.VMEM_SHARED((NP, DH), jnp.float32),
    ],
    compiler_params=pltpu.CompilerParams(use_tc_tiling_on_sc=False),
)
def _sc_msgpass(h0_hbm, src2_hbm, dst_hbm, ewp_hbm, deg_hbm, znd_hbm,
                h1_out, h2_out,
                sv0, dv0, ev0, sv1, dv1, ev1, rowsA, rowsB, invv,
                semA, semB, agg_sh):
    c = lax.axis_index("c")
    s = lax.axis_index("s")

    def mul_rows(rows, ev):
        def mul_body(k16, _):
            kb = k16 * 16
            w16 = ev[pl.ds(kb, 16)]
            for j in range(16):
                wk = w16[j]
                for d in range(DH // 16):
                    rows[kb + j, pl.ds(d * 16, 16)] = (
                        rows[kb + j, pl.ds(d * 16, 16)] * wk)
            return 0
        lax.fori_loop(0, K // 16, mul_body, 0)

    for step in range(2):
        h_src = h0_hbm if step == 0 else h1_out
        h_dst = h1_out if step == 0 else h2_out

        def pass_body(p, _):
            gi = p // 2
            fi = p % 2
            g = 2 * c + gi
            nbase = fi * (NG * NP) + g * NP + s * NPT
            abase = g * NP + s * NPT
            ebase = fi * (NG * E) + g * E + s * EPT
            dbase = g * E + s * EPT

            def load_idx(i, sv, dv, ev):
                pltpu.sync_copy(src2_hbm.at[pl.ds(ebase + i * K, K)],
                                sv.at[0])
                pltpu.sync_copy(dst_hbm.at[pl.ds(dbase + i * K, K)],
                                dv.at[0])
                pltpu.sync_copy(ewp_hbm.at[pl.ds(dbase + i * K, K)], ev)

            pltpu.sync_copy(deg_hbm.at[pl.ds(abase, NPT)], invv)

            def inv_body(i, _):
                d16 = invv[pl.ds(i * 16, 16)]
                invv[pl.ds(i * 16, 16)] = 1.0 / jnp.maximum(d16, 1.0)
                return 0
            lax.fori_loop(0, NPT // 16, inv_body, 0)

            if True:
                pltpu.sync_copy(znd_hbm.at[pl.ds(s * NPT, NPT)],
                                agg_sh.at[pl.ds(s * NPT, NPT)])
                plsc.subcore_barrier()

                load_idx(0, sv0, dv0, ev0)
                pltpu.async_copy(h_src.at[sv0.at[0]], rowsA, semA)
                load_idx(1, sv1, dv1, ev1)

                def body(t, _):
                    pltpu.make_async_copy(h_src.at[sv0.at[0]], rowsA,
                                          semA).wait()
                    pltpu.async_copy(h_src.at[sv1.at[0]], rowsB, semB)
                    mul_rows(rowsA, ev0)
                    pltpu.sync_copy(rowsA, agg_sh.at[dv0.at[0]], add=True)

                    @pl.when(t < NCH // 2 - 1)
                    def _():
                        load_idx(2 * t + 2, sv0, dv0, ev0)
                        pltpu.async_copy(h_src.at[sv0.at[0]], rowsA, semA)

                    pltpu.make_async_copy(h_src.at[sv1.at[0]], rowsB,
                                          semB).wait()
                    mul_rows(rowsB, ev1)
                    pltpu.sync_copy(rowsB, agg_sh.at[dv1.at[0]], add=True)

                    @pl.when(t < NCH // 2 - 1)
                    def _():
                        load_idx(2 * t + 3, sv1, dv1, ev1)
                    return 0
                lax.fori_loop(0, NCH // 2, body, 0)
                plsc.subcore_barrier()

                for r in range(2):
                    rb = r * HCH
                    pltpu.sync_copy(agg_sh.at[pl.ds(s * NPT + rb, HCH)],
                                    rowsA.at[pl.ds(0, HCH)])
                    pltpu.sync_copy(h_src.at[pl.ds(nbase + rb, HCH)],
                                    rowsB.at[pl.ds(0, HCH)])

                    def upd_body(i2, _):
                        ib = i2 * 16
                        iv16 = invv[pl.ds(rb + ib, 16)]
                        for j in range(16):
                            sj = iv16[j]
                            for d in range(DH // 16):
                                rowsB[ib + j, pl.ds(d * 16, 16)] = (
                                    rowsB[ib + j, pl.ds(d * 16, 16)]
                                    + rowsA[ib + j, pl.ds(d * 16, 16)] * sj)
                        return 0
                    lax.fori_loop(0, HCH // 16, upd_body, 0)
                    pltpu.sync_copy(rowsB.at[pl.ds(0, HCH)],
                                    h_dst.at[pl.ds(nbase + rb, HCH)])
                plsc.subcore_barrier()
            return 0
        lax.fori_loop(0, 4, pass_body, 0)


_PBLK = 1024


def _tc_proj(x, w, b):
    def body(x_ref, w_ref, b_ref, oa_ref, ob_ref):
        h = (jnp.dot(x_ref[0], w_ref[0], preferred_element_type=jnp.float32)
             + b_ref[0])
        oa_ref[0] = h[:, :DH]
        ob_ref[0] = h[:, DH:]
    return pl.pallas_call(
        body,
        grid=(NG, NP // _PBLK),
        in_specs=[
            pl.BlockSpec((1, _PBLK, D), lambda g, nb: (g, nb, 0)),
            pl.BlockSpec((1, D, D), lambda g, nb: (g, 0, 0)),
            pl.BlockSpec((1, 1, D), lambda g, nb: (g, 0, 0)),
        ],
        out_specs=[
            pl.BlockSpec((1, _PBLK, DH), lambda g, nb: (g, nb, 0)),
            pl.BlockSpec((1, _PBLK, DH), lambda g, nb: (g, nb, 0)),
        ],
        out_shape=[
            jax.ShapeDtypeStruct((NG, NP, DH), jnp.float32),
            jax.ShapeDtypeStruct((NG, NP, DH), jnp.float32),
        ],
    )(x, w, b)


def _tc_readout(h2in, seg3, cw, cb):
    def body(h_ref, s_ref, w_ref, b_ref, o_ref):
        h2 = h_ref[0]
        seg = s_ref[0]
        io = lax.broadcasted_iota(jnp.int32, (NP, B), 1)
        onehot = (seg == io).astype(jnp.float32)
        counts = jnp.sum(onehot, axis=0)
        doc = lax.dot_general(onehot, h2, (((0,), (0,)), ((), ())))
        doc = doc / jnp.maximum(counts, 1.0)[:, None]
        a = jnp.maximum(doc, 0.0)
        o_ref[0] = (
            jnp.dot(a, w_ref[0], preferred_element_type=jnp.float32)
            + b_ref[0]
        )
    return pl.pallas_call(
        body,
        grid=(NG,),
        in_specs=[
            pl.BlockSpec((1, NP, D), lambda g: (g, 0, 0)),
            pl.BlockSpec((1, NP, 1), lambda g: (g, 0, 0)),
            pl.BlockSpec((1, D, C), lambda g: (g, 0, 0)),
            pl.BlockSpec((1, 1, C), lambda g: (g, 0, 0)),
        ],
        out_specs=pl.BlockSpec((1, B, C), lambda g: (g, 0, 0)),
        out_shape=jax.ShapeDtypeStruct((NG, B, C), jnp.float32),
    )(h2in, seg3, cw, cb)


def _tc_head(logits, fw, fb):
    def body(l_ref, w_ref, b_ref, p_ref, f_ref):
        wm = jnp.mean(w_ref[...], axis=0)
        bm = jnp.mean(b_ref[...], axis=0)
        acc = jnp.zeros((B, C), jnp.float32)
        for g in range(NG):
            acc = acc + lax.dot_general(
                l_ref[g], wm[:, :, g], (((1,), (1,)), ((), ())),
                preferred_element_type=jnp.float32)
        fused = acc + bm[None, :]
        m = jnp.max(fused, axis=-1, keepdims=True)
        e = jnp.exp(fused - m)
        p = e / jnp.sum(e, axis=-1, keepdims=True)
        p_ref[...] = p
        f_ref[...] = fused
    return pl.pallas_call(
        body,
        in_specs=[
            pl.BlockSpec(logits.shape, lambda: (0, 0, 0)),
            pl.BlockSpec(fw.shape, lambda: (0, 0, 0, 0)),
            pl.BlockSpec(fb.shape, lambda: (0, 0)),
        ],
        out_specs=[
            pl.BlockSpec((B, C), lambda: (0, 0)),
            pl.BlockSpec((B, C), lambda: (0, 0)),
        ],
        out_shape=[
            jax.ShapeDtypeStruct((B, C), jnp.float32),
            jax.ShapeDtypeStruct((B, C), jnp.float32),
        ],
    )(logits, fw, fb)


def kernel(node_indices_batch, edge_index_batch, edge_weights_batch,
           node_graph_ids_batch, emb_table, edge_params, proj_W, proj_b,
           clf_W, clf_b, fusion_W, fusion_b):
    ni_pad = jnp.concatenate(
        [node_indices_batch,
         jnp.zeros((NG, NP - N), jnp.int32)], axis=1).reshape(-1)
    src = edge_index_batch[:, 0, :]
    dst = edge_index_batch[:, 1, :]
    src_glob = (src + (jnp.arange(NG, dtype=jnp.int32) * NP)[:, None]).reshape(-1)
    src2 = jnp.concatenate([src_glob, src_glob + NG * NP])
    dst_flat = dst.reshape(-1)
    ew_flat = edge_weights_batch.reshape(-1)
    params_rep = jnp.broadcast_to(edge_params[:, None], (NG, 16)).reshape(-1)
    zeros_n = jnp.zeros((NP,), jnp.float32)
    zeros_nd = jnp.zeros((NP, DH), jnp.float32)
    seg3 = jnp.concatenate(
        [node_graph_ids_batch,
         jnp.full((NG, NP - N), B, jnp.int32)],
        axis=1)[:, :, None]

    x_flat, ewp_flat, deg_flat = _sc_prep(ni_pad, emb_table, dst_flat, ew_flat,
                                          params_rep, zeros_n)

    h0a, h0b = _tc_proj(x_flat.reshape(NG, NP, D), proj_W,
                        proj_b[:, None, :])

    h0_2 = jnp.concatenate([h0a.reshape(NG * NP, DH),
                            h0b.reshape(NG * NP, DH)], axis=0)
    _, h2_2 = _sc_msgpass(h0_2, src2, dst_flat, ewp_flat, deg_flat,
                          zeros_nd)
    h2 = jnp.concatenate([h2_2[:NG * NP], h2_2[NG * NP:]], axis=1)

    logits = _tc_readout(h2.reshape(NG, NP, D), seg3, clf_W,
                         clf_b[:, None, :])

    predictions, fused_logits = _tc_head(logits, fusion_W, fusion_b)
    return (predictions, fused_logits)
